# baseline (speedup 1.0000x reference)
import functools

import jax
import jax.numpy as jnp
from jax.experimental import pallas as pl
from jax.experimental.pallas import tpu as pltpu

NQ = 8
K = 1024
D = 256
BN = 512


def _row_sum_sq(x):
    s = x * x
    t = s[:, :128] + s[:, 128:]
    acc = t[:, 0:8]
    for i in range(1, 16):
        acc = acc + t[:, 8 * i:8 * i + 8]
    a4 = acc[:, 0:4] + acc[:, 4:8]
    a2 = a4[:, 0:2] + a4[:, 2:4]
    return a2[:, 0:1] + a2[:, 1:2]


def _rvq_kernel(z_ref, cb_ref, hi_ref, mid_ref, lo_ref, out_ref, idx_ref,
                w2_ref):
    @pl.when(pl.program_id(0) == 0)
    def _():
        for q in range(NQ):
            w2_ref[q:q + 1, :] = jnp.transpose(_row_sum_sq(cb_ref[q]))

    r = z_ref[...]
    acc = jnp.zeros_like(r)
    iota = jax.lax.broadcasted_iota(jnp.int32, (BN, K), 1)
    for q in range(NQ):
        w = cb_ref[q]
        w2r = w2_ref[q:q + 1, :]
        r2 = _row_sum_sq(r)
        mm2 = jax.lax.dot_general(
            r + r, w, (((1,), (1,)), ((), ())),
            preferred_element_type=jnp.float32)
        scores = (r2 - mm2) + w2r
        m = jnp.min(scores, axis=1, keepdims=True)
        idx = jnp.min(jnp.where(scores == m, iota, K), axis=1)
        idx_ref[q, :] = idx.astype(jnp.int32)
        onehot = (iota == idx[:, None]).astype(jnp.float32).astype(jnp.bfloat16)
        quant = jnp.dot(onehot, hi_ref[q], preferred_element_type=jnp.float32)
        quant = quant + jnp.dot(onehot, mid_ref[q],
                                preferred_element_type=jnp.float32)
        quant = quant + jnp.dot(onehot, lo_ref[q],
                                preferred_element_type=jnp.float32)
        acc = acc + quant
        r = r - quant
    out_ref[...] = acc


@functools.partial(jax.jit, static_argnames=("interpret",))
def _rvq(z_flat, codebooks, interpret=False):
    n = z_flat.shape[0]
    grid = (n // BN,)
    def _trunc_plane(x):
        bits = jax.lax.bitcast_convert_type(x, jnp.uint32)
        hi32 = jax.lax.bitcast_convert_type(
            bits & jnp.uint32(0xFFFF0000), jnp.float32)
        plane = jax.lax.bitcast_convert_type(
            (bits >> jnp.uint32(16)).astype(jnp.uint16), jnp.bfloat16)
        return plane, x - hi32

    cb_hi, rem = _trunc_plane(codebooks)
    cb_mid, rem2 = _trunc_plane(rem)
    cb_lo, _ = _trunc_plane(rem2)
    cbspec = pl.BlockSpec((NQ, K, D), lambda i: (0, 0, 0))
    out, idx_t = pl.pallas_call(
        _rvq_kernel,
        grid=grid,
        in_specs=[
            pl.BlockSpec((BN, D), lambda i: (i, 0)),
            cbspec, cbspec, cbspec, cbspec,
        ],
        out_specs=[
            pl.BlockSpec((BN, D), lambda i: (i, 0)),
            pl.BlockSpec((NQ, BN), lambda i: (0, i)),
        ],
        out_shape=[
            jax.ShapeDtypeStruct((n, D), jnp.float32),
            jax.ShapeDtypeStruct((NQ, n), jnp.int32),
        ],
        scratch_shapes=[pltpu.VMEM((NQ, K), jnp.float32)],
        interpret=interpret,
    )(z_flat, codebooks, cb_hi, cb_mid, cb_lo)
    return out, idx_t


def kernel(z_e, codebooks):
    b, l, d = z_e.shape
    out, idx_t = _rvq(z_e.reshape(-1, d), codebooks)
    return out.reshape(b, l, d), idx_t.T.reshape(b, l, NQ)


# speedup vs baseline: 1.3279x; 1.3279x over previous
import functools

import jax
import jax.numpy as jnp
from jax.experimental import pallas as pl
from jax.experimental.pallas import tpu as pltpu

NQ = 8
K = 1024
D = 256
BN = 1024


def _row_sum_sq(x):
    s = x * x
    t = s[:, :128] + s[:, 128:]
    acc = t[:, 0:8]
    for i in range(1, 16):
        acc = acc + t[:, 8 * i:8 * i + 8]
    a4 = acc[:, 0:4] + acc[:, 4:8]
    a2 = a4[:, 0:2] + a4[:, 2:4]
    return a2[:, 0:1] + a2[:, 1:2]


def _rvq_kernel(z_ref, cb_ref, hi_ref, mid_ref, lo_ref, out_ref, idx_ref):
    r = z_ref[...]
    acc = jnp.zeros_like(r)
    iota = jax.lax.broadcasted_iota(jnp.int32, (BN, K), 1)
    for q in range(NQ):
        w = cb_ref[q]
        w2r = jnp.transpose(_row_sum_sq(w))
        r2 = _row_sum_sq(r)
        mm2 = jax.lax.dot_general(
            r + r, w, (((1,), (1,)), ((), ())),
            preferred_element_type=jnp.float32)
        scores = (r2 - mm2) + w2r
        m = jnp.min(scores, axis=1, keepdims=True)
        idx = jnp.min(jnp.where(scores == m, iota, K), axis=1)
        idx_ref[q, :] = idx.astype(jnp.int32)
        onehot = (iota == idx[:, None]).astype(jnp.float32).astype(jnp.bfloat16)
        quant = jnp.dot(onehot, hi_ref[q], preferred_element_type=jnp.float32)
        quant = quant + jnp.dot(onehot, mid_ref[q],
                                preferred_element_type=jnp.float32)
        quant = quant + jnp.dot(onehot, lo_ref[q],
                                preferred_element_type=jnp.float32)
        acc = acc + quant
        r = r - quant
    out_ref[...] = acc


@functools.partial(jax.jit, static_argnames=("interpret",))
def _rvq(z_flat, codebooks, interpret=False):
    n = z_flat.shape[0]
    grid = (n // BN,)
    def _trunc_plane(x):
        bits = jax.lax.bitcast_convert_type(x, jnp.uint32)
        hi32 = jax.lax.bitcast_convert_type(
            bits & jnp.uint32(0xFFFF0000), jnp.float32)
        plane = jax.lax.bitcast_convert_type(
            (bits >> jnp.uint32(16)).astype(jnp.uint16), jnp.bfloat16)
        return plane, x - hi32

    cb_hi, rem = _trunc_plane(codebooks)
    cb_mid, rem2 = _trunc_plane(rem)
    cb_lo, _ = _trunc_plane(rem2)
    cbspec = pl.BlockSpec((NQ, K, D), lambda i: (0, 0, 0))
    out, idx_t = pl.pallas_call(
        _rvq_kernel,
        grid=grid,
        in_specs=[
            pl.BlockSpec((BN, D), lambda i: (i, 0)),
            cbspec, cbspec, cbspec, cbspec,
        ],
        out_specs=[
            pl.BlockSpec((BN, D), lambda i: (i, 0)),
            pl.BlockSpec((NQ, BN), lambda i: (0, i)),
        ],
        out_shape=[
            jax.ShapeDtypeStruct((n, D), jnp.float32),
            jax.ShapeDtypeStruct((NQ, n), jnp.int32),
        ],
        interpret=interpret,
    )(z_flat, codebooks, cb_hi, cb_mid, cb_lo)
    return out, idx_t


def kernel(z_e, codebooks):
    b, l, d = z_e.shape
    out, idx_t = _rvq(z_e.reshape(-1, d), codebooks)
    return out.reshape(b, l, d), idx_t.T.reshape(b, l, NQ)
